# baseline (speedup 1.0000x reference)
import jax
import jax.numpy as jnp
from jax import lax
from jax.experimental import pallas as pl
from jax.experimental.pallas import tpu as pltpu
from jax.experimental.pallas import tpu_sc as plsc

_N_HD = 20000
_N_HD_PAD = 20480
_NNZ = 3
_N_SMPL = 6890
_B = 256
_NC = 2
_NS = 16
_NW = _NC * _NS
_B_PER_W = _B // _NW
_R = 1024
_NCH = _N_HD_PAD // _R
_GRP = _R // 16

_INV_A2 = 1.0 / 0.15


def _sc_body(y_hbm, cv_hbm, wv_hbm, h_hbm,
             table, cba, cbb, wba, wbb, oba, obb,
             si0, si1, so0, so1):
    wid = lax.axis_index("s") * _NC + lax.axis_index("c")
    b0 = wid * _B_PER_W

    cbufs = (cba, cbb)
    wbufs = (wba, wbb)
    obufs = (oba, obb)
    sems_in = (si0, si1)
    sems_out = (so0, so1)

    def fire_in(ch):
        par = ch % 2
        off = ch * _R * _NNZ
        return [pltpu.async_copy(cv_hbm.at[pl.ds(off, _R * _NNZ)],
                                 cbufs[par], sems_in[par]),
                pltpu.async_copy(wv_hbm.at[pl.ds(off, _R * _NNZ)],
                                 wbufs[par], sems_in[par])]

    pend_in = {0: fire_in(0)}
    pltpu.sync_copy(y_hbm.at[pl.ds(b0, _B_PER_W)], table)
    pend_out = {}

    iota3 = lax.iota(jnp.int32, 16) * _NNZ

    for ch in range(_NCH):
        par = ch % 2
        cb, wb, ob = cbufs[par], wbufs[par], obufs[par]
        if ch + 1 < _NCH:
            pend_in[ch + 1] = fire_in(ch + 1)
        for d in pend_in.pop(ch):
            d.wait()
        if ch - 2 in pend_out:
            for d in pend_out.pop(ch - 2):
                d.wait()

        @plsc.parallel_loop(0, _GRP, unroll=1)
        def _group(g):
            i0 = iota3 + g * (16 * _NNZ)
            i1 = i0 + 1
            i2 = i0 + 2
            c0 = plsc.load_gather(cb, [i0])
            c1 = plsc.load_gather(cb, [i1])
            c2 = plsc.load_gather(cb, [i2])
            w0 = plsc.load_gather(wb, [i0])
            w1 = plsc.load_gather(wb, [i1])
            w2 = plsc.load_gather(wb, [i2])
            base = g * 16
            for b in range(_B_PER_W):
                bi = jnp.full((16,), b, jnp.int32)
                g0 = plsc.load_gather(table, [bi, c0])
                g1 = plsc.load_gather(table, [bi, c1])
                g2 = plsc.load_gather(table, [bi, c2])
                ob[b, pl.ds(base, 16)] = g0 * w0 + g1 * w1 + g2 * w2

        pend_out[ch] = [pltpu.async_copy(
            ob,
            h_hbm.at[pl.ds(b0, _B_PER_W), pl.ds(ch * _R, _R)],
            sems_out[par])]

    for ch in sorted(pend_out):
        for d in pend_out[ch]:
            d.wait()


def _tc_body(h_ref, o_ref):
    h = h_ref[:, : _N_HD]
    t = jnp.tanh(h * _INV_A2)
    o_ref[...] = jnp.where(h < 0.0, 10.0, 1.0) * (t * t)


@jax.jit
def kernel(vertices, op_values, op_rows, op_cols):
    del op_rows
    y = vertices[:, :, 1]
    pad = (_N_HD_PAD - _N_HD) * _NNZ
    cv = jnp.concatenate([op_cols.astype(jnp.int32),
                          jnp.zeros((pad,), jnp.int32)])
    wv = jnp.concatenate([op_values.astype(jnp.float32),
                          jnp.zeros((pad,), jnp.float32)])

    mesh = plsc.VectorSubcoreMesh(core_axis_name="c", subcore_axis_name="s")
    sc_fn = pl.kernel(
        _sc_body,
        out_type=jax.ShapeDtypeStruct((_B, _N_HD_PAD), jnp.float32),
        mesh=mesh,
        compiler_params=pltpu.CompilerParams(
            use_tc_tiling_on_sc=True, needs_layout_passes=False),
        scratch_types=[
            pltpu.VMEM((_B_PER_W, _N_SMPL), jnp.float32),
            pltpu.VMEM((_R * _NNZ,), jnp.int32),
            pltpu.VMEM((_R * _NNZ,), jnp.int32),
            pltpu.VMEM((_R * _NNZ,), jnp.float32),
            pltpu.VMEM((_R * _NNZ,), jnp.float32),
            pltpu.VMEM((_B_PER_W, _R), jnp.float32),
            pltpu.VMEM((_B_PER_W, _R), jnp.float32),
            pltpu.SemaphoreType.DMA,
            pltpu.SemaphoreType.DMA,
            pltpu.SemaphoreType.DMA,
            pltpu.SemaphoreType.DMA,
        ],
    )
    h = sc_fn(y, cv, wv)

    blk = 32
    out = pl.pallas_call(
        _tc_body,
        out_shape=jax.ShapeDtypeStruct((_B, _N_HD), jnp.float32),
        grid=(_B // blk,),
        in_specs=[pl.BlockSpec((blk, _N_HD_PAD), lambda i: (i, 0))],
        out_specs=pl.BlockSpec((blk, _N_HD), lambda i: (i, 0)),
    )(h)
    return out


# speedup vs baseline: 1.0865x; 1.0865x over previous
import jax
import jax.numpy as jnp
from jax import lax
from jax.experimental import pallas as pl
from jax.experimental.pallas import tpu as pltpu
from jax.experimental.pallas import tpu_sc as plsc

_N_HD = 20000
_N_HD_PAD = 20480
_NNZ = 3
_N_SMPL = 6890
_B = 256
_NC = 2
_NS = 16
_NW = _NC * _NS
_B_PER_W = _B // _NW
_R = 2048
_NCH = _N_HD_PAD // _R
_GRP = _R // 16

_INV_A2 = 1.0 / 0.15


def _sc_body(y_hbm, cv_hbm, wv_hbm, h_hbm,
             table, cba, cbb, wba, wbb, oba, obb,
             si0, si1, so0, so1):
    wid = lax.axis_index("s") * _NC + lax.axis_index("c")
    b0 = wid * _B_PER_W

    cbufs = (cba, cbb)
    wbufs = (wba, wbb)
    obufs = (oba, obb)
    sems_in = (si0, si1)
    sems_out = (so0, so1)

    def fire_in(ch):
        par = ch % 2
        off = ch * _R * _NNZ
        return [pltpu.async_copy(cv_hbm.at[pl.ds(off, _R * _NNZ)],
                                 cbufs[par], sems_in[par]),
                pltpu.async_copy(wv_hbm.at[pl.ds(off, _R * _NNZ)],
                                 wbufs[par], sems_in[par])]

    pend_in = {0: fire_in(0)}
    pltpu.sync_copy(y_hbm.at[pl.ds(b0, _B_PER_W)], table)
    pend_out = {}

    iota3 = lax.iota(jnp.int32, 16) * _NNZ

    for ch in range(_NCH):
        par = ch % 2
        cb, wb, ob = cbufs[par], wbufs[par], obufs[par]
        if ch + 1 < _NCH:
            pend_in[ch + 1] = fire_in(ch + 1)
        for d in pend_in.pop(ch):
            d.wait()
        if ch - 2 in pend_out:
            for d in pend_out.pop(ch - 2):
                d.wait()

        @plsc.parallel_loop(0, _GRP, unroll=1)
        def _group(g):
            i0 = iota3 + g * (16 * _NNZ)
            i1 = i0 + 1
            i2 = i0 + 2
            c0 = plsc.load_gather(cb, [i0])
            c1 = plsc.load_gather(cb, [i1])
            c2 = plsc.load_gather(cb, [i2])
            w0 = plsc.load_gather(wb, [i0])
            w1 = plsc.load_gather(wb, [i1])
            w2 = plsc.load_gather(wb, [i2])
            base = g * 16
            for b in range(_B_PER_W):
                bi = jnp.full((16,), b, jnp.int32)
                g0 = plsc.load_gather(table, [bi, c0])
                g1 = plsc.load_gather(table, [bi, c1])
                g2 = plsc.load_gather(table, [bi, c2])
                ob[b, pl.ds(base, 16)] = g0 * w0 + g1 * w1 + g2 * w2

        pend_out[ch] = [pltpu.async_copy(
            ob,
            h_hbm.at[pl.ds(b0, _B_PER_W), pl.ds(ch * _R, _R)],
            sems_out[par])]

    for ch in sorted(pend_out):
        for d in pend_out[ch]:
            d.wait()


def _tc_body(h_ref, o_ref):
    h = h_ref[:, : _N_HD]
    t = jnp.tanh(h * _INV_A2)
    o_ref[...] = jnp.where(h < 0.0, 10.0, 1.0) * (t * t)


@jax.jit
def kernel(vertices, op_values, op_rows, op_cols):
    del op_rows
    y = vertices[:, :, 1]
    pad = (_N_HD_PAD - _N_HD) * _NNZ
    cv = jnp.concatenate([op_cols.astype(jnp.int32),
                          jnp.zeros((pad,), jnp.int32)])
    wv = jnp.concatenate([op_values.astype(jnp.float32),
                          jnp.zeros((pad,), jnp.float32)])

    mesh = plsc.VectorSubcoreMesh(core_axis_name="c", subcore_axis_name="s")
    sc_fn = pl.kernel(
        _sc_body,
        out_type=jax.ShapeDtypeStruct((_B, _N_HD_PAD), jnp.float32),
        mesh=mesh,
        compiler_params=pltpu.CompilerParams(
            use_tc_tiling_on_sc=True, needs_layout_passes=False),
        scratch_types=[
            pltpu.VMEM((_B_PER_W, _N_SMPL), jnp.float32),
            pltpu.VMEM((_R * _NNZ,), jnp.int32),
            pltpu.VMEM((_R * _NNZ,), jnp.int32),
            pltpu.VMEM((_R * _NNZ,), jnp.float32),
            pltpu.VMEM((_R * _NNZ,), jnp.float32),
            pltpu.VMEM((_B_PER_W, _R), jnp.float32),
            pltpu.VMEM((_B_PER_W, _R), jnp.float32),
            pltpu.SemaphoreType.DMA,
            pltpu.SemaphoreType.DMA,
            pltpu.SemaphoreType.DMA,
            pltpu.SemaphoreType.DMA,
        ],
    )
    h = sc_fn(y, cv, wv)

    blk = 32
    out = pl.pallas_call(
        _tc_body,
        out_shape=jax.ShapeDtypeStruct((_B, _N_HD), jnp.float32),
        grid=(_B // blk,),
        in_specs=[pl.BlockSpec((blk, _N_HD_PAD), lambda i: (i, 0))],
        out_specs=pl.BlockSpec((blk, _N_HD), lambda i: (i, 0)),
    )(h)
    return out
